# Initial kernel scaffold; baseline (speedup 1.0000x reference)
import jax
import jax.numpy as jnp
from jax.experimental import pallas as pl


def kernel(input_ids, table, W_trans, b_trans):
    raise NotImplementedError("write your pallas kernel here")


# speedup vs baseline: 9.3319x; 9.3319x over previous
import functools

import jax
import jax.numpy as jnp
from jax import lax
from jax.experimental import pallas as pl
from jax.experimental.pallas import tpu as pltpu
from jax.experimental.pallas import tpu_sc as plsc

L, B, C = 200, 1024, 8
VOCAB, D_SUB, D_EMBED = 100000, 32, 128
N_TOK = L * B
N_ROWS = N_TOK * C

NC, NS = 2, 16
NW = NC * NS
ROWS_PER_W = N_ROWS // NW
CHUNK = 1024
N_CHUNKS = ROWS_PER_W // CHUNK


def _gather_body(idx_hbm, table_hbm, out_hbm, idx_v, rows_v, sem):
    wid = lax.axis_index("s") * NC + lax.axis_index("c")
    base_w = wid * ROWS_PER_W

    def step(i, carry):
        base = base_w + i * CHUNK
        pltpu.sync_copy(idx_hbm.at[pl.ds(base, CHUNK)], idx_v)
        pltpu.async_copy(table_hbm.at[idx_v], rows_v, sem).wait()
        pltpu.sync_copy(rows_v, out_hbm.at[pl.ds(base, CHUNK)])
        return carry

    lax.fori_loop(0, N_CHUNKS, step, 0)


_sc_gather = functools.partial(
    pl.kernel,
    out_type=jax.ShapeDtypeStruct((N_ROWS, D_SUB), jnp.float32),
    mesh=plsc.VectorSubcoreMesh(core_axis_name="c", subcore_axis_name="s"),
    scratch_types=[
        pltpu.VMEM((CHUNK,), jnp.int32),
        pltpu.VMEM((CHUNK, D_SUB), jnp.float32),
        pltpu.SemaphoreType.DMA,
    ],
    compiler_params=pltpu.CompilerParams(use_tc_tiling_on_sc=False),
)(_gather_body)


def _mm_body(x_ref, w_ref, b_ref, o_ref):
    o_ref[...] = (
        jnp.dot(x_ref[...], w_ref[...], preferred_element_type=jnp.float32)
        + b_ref[0, :]
    )


MM_BLOCK = 2048


def _tc_matmul(x, w, b):
    n = x.shape[0]
    d_in = x.shape[1]
    d_out = w.shape[1]
    return pl.pallas_call(
        _mm_body,
        grid=(n // MM_BLOCK,),
        in_specs=[
            pl.BlockSpec((MM_BLOCK, d_in), lambda i: (i, 0)),
            pl.BlockSpec((d_in, d_out), lambda i: (0, 0)),
            pl.BlockSpec((1, d_out), lambda i: (0, 0)),
        ],
        out_specs=pl.BlockSpec((MM_BLOCK, d_out), lambda i: (i, 0)),
        out_shape=jax.ShapeDtypeStruct((n, d_out), jnp.float32),
    )(x, w, b)


@jax.jit
def kernel(input_ids, table, W_trans, b_trans):
    idx_flat = input_ids.reshape(N_ROWS)
    gathered = _sc_gather(idx_flat, table)
    x = gathered.reshape(N_TOK, C * D_SUB)
    out = _tc_matmul(x, W_trans.T, b_trans.reshape(1, D_EMBED))
    return out.reshape(L, B, D_EMBED)


# speedup vs baseline: 13.4043x; 1.4364x over previous
import functools

import jax
import jax.numpy as jnp
from jax import lax
from jax.experimental import pallas as pl
from jax.experimental.pallas import tpu as pltpu
from jax.experimental.pallas import tpu_sc as plsc

L, B, C = 200, 1024, 8
VOCAB, D_SUB, D_EMBED = 100000, 32, 128
N_TOK = L * B
N_ROWS = N_TOK * C
N_OUT = N_ROWS // 4

NC, NS = 2, 16
NW = NC * NS
ROWS_PER_W = N_ROWS // NW
CHUNK = 1024
OUT_CHUNK = CHUNK // 4
N_CHUNKS = ROWS_PER_W // CHUNK


def _gather_body(idx_hbm, table_hbm, out_hbm, idx_v, rows_v, sem):
    wid = lax.axis_index("s") * NC + lax.axis_index("c")
    base_w = wid * ROWS_PER_W

    def step(i, carry):
        base = base_w + i * CHUNK
        pltpu.sync_copy(idx_hbm.at[pl.ds(base, CHUNK)], idx_v)
        pltpu.async_copy(table_hbm.at[idx_v], rows_v, sem).wait()
        out_base = base // 4
        for j in range(4):
            pltpu.sync_copy(
                rows_v.at[pl.ds(j * OUT_CHUNK, OUT_CHUNK), :],
                out_hbm.at[pl.ds(out_base, OUT_CHUNK), pl.ds(j * D_SUB, D_SUB)],
            )
        return carry

    lax.fori_loop(0, N_CHUNKS, step, 0)


_sc_gather = functools.partial(
    pl.kernel,
    out_type=jax.ShapeDtypeStruct((N_OUT, 128), jnp.float32),
    mesh=plsc.VectorSubcoreMesh(core_axis_name="c", subcore_axis_name="s"),
    scratch_types=[
        pltpu.VMEM((CHUNK,), jnp.int32),
        pltpu.VMEM((CHUNK, D_SUB), jnp.float32),
        pltpu.SemaphoreType.DMA,
    ],
    compiler_params=pltpu.CompilerParams(use_tc_tiling_on_sc=False),
)(_gather_body)


MM_TOK = 1024


def _mm_body(x_ref, w_ref, b_ref, o_ref):
    x0 = x_ref[0:MM_TOK, :]
    x1 = x_ref[MM_TOK : 2 * MM_TOK, :]
    w0 = w_ref[0:128, :]
    w1 = w_ref[128:256, :]
    o_ref[...] = (
        jnp.dot(x0, w0, preferred_element_type=jnp.float32)
        + jnp.dot(x1, w1, preferred_element_type=jnp.float32)
        + b_ref[0, :]
    )


def _tc_matmul(x, w, b):
    return pl.pallas_call(
        _mm_body,
        grid=(N_TOK // MM_TOK,),
        in_specs=[
            pl.BlockSpec((2 * MM_TOK, 128), lambda i: (i, 0)),
            pl.BlockSpec((256, D_EMBED), lambda i: (0, 0)),
            pl.BlockSpec((1, D_EMBED), lambda i: (0, 0)),
        ],
        out_specs=pl.BlockSpec((MM_TOK, D_EMBED), lambda i: (i, 0)),
        out_shape=jax.ShapeDtypeStruct((N_TOK, D_EMBED), jnp.float32),
    )(x, w, b)


@jax.jit
def kernel(input_ids, table, W_trans, b_trans):
    idx_flat = (
        input_ids.reshape(L, 4, 256, 2, 4).transpose(0, 3, 1, 4, 2).reshape(N_ROWS)
    )
    gathered = _sc_gather(idx_flat, table)
    out = _tc_matmul(gathered, W_trans.T, b_trans.reshape(1, D_EMBED))
    return out.reshape(L, B, D_EMBED)


# speedup vs baseline: 14.5766x; 1.0875x over previous
import functools

import jax
import jax.numpy as jnp
from jax import lax
from jax.experimental import pallas as pl
from jax.experimental.pallas import tpu as pltpu
from jax.experimental.pallas import tpu_sc as plsc

L, B, C = 200, 1024, 8
VOCAB, D_SUB, D_EMBED = 100000, 32, 128
N_TOK = L * B
N_ROWS = N_TOK * C
N_OUT = N_ROWS // 4

NC, NS = 2, 16
NW = NC * NS
ROWS_PER_W = N_ROWS // NW
CHUNK = 1024
OUT_CHUNK = CHUNK // 4
N_CHUNKS = ROWS_PER_W // CHUNK


def _writeback(rows_v, out_hbm, out_base):
    for j in range(4):
        pltpu.sync_copy(
            rows_v.at[pl.ds(j * OUT_CHUNK, OUT_CHUNK), :],
            out_hbm.at[pl.ds(out_base, OUT_CHUNK), pl.ds(j * D_SUB, D_SUB)],
        )


def _gather_body(idx_hbm, table_hbm, out_hbm, idx_a, idx_b, rows_a, rows_b,
                 sem_a, sem_b):
    wid = lax.axis_index("s") * NC + lax.axis_index("c")
    base_w = wid * ROWS_PER_W
    n_half = N_CHUNKS // 2

    def load_and_fire(chunk, idx_v, rows_v, sem):
        pltpu.sync_copy(idx_hbm.at[pl.ds(base_w + chunk * CHUNK, CHUNK)], idx_v)
        pltpu.async_copy(table_hbm.at[idx_v], rows_v, sem)

    load_and_fire(0, idx_a, rows_a, sem_a)

    def step(i, carry):
        load_and_fire(2 * i + 1, idx_b, rows_b, sem_b)
        pltpu.make_async_copy(table_hbm.at[idx_a], rows_a, sem_a).wait()
        _writeback(rows_a, out_hbm, (base_w + 2 * i * CHUNK) // 4)

        @pl.when(i < n_half - 1)
        def _():
            load_and_fire(2 * i + 2, idx_a, rows_a, sem_a)

        pltpu.make_async_copy(table_hbm.at[idx_b], rows_b, sem_b).wait()
        _writeback(rows_b, out_hbm, (base_w + (2 * i + 1) * CHUNK) // 4)
        return carry

    lax.fori_loop(0, n_half, step, 0)


_sc_gather = functools.partial(
    pl.kernel,
    out_type=jax.ShapeDtypeStruct((N_OUT, 128), jnp.float32),
    mesh=plsc.VectorSubcoreMesh(core_axis_name="c", subcore_axis_name="s"),
    scratch_types=[
        pltpu.VMEM((CHUNK,), jnp.int32),
        pltpu.VMEM((CHUNK,), jnp.int32),
        pltpu.VMEM((CHUNK, D_SUB), jnp.float32),
        pltpu.VMEM((CHUNK, D_SUB), jnp.float32),
        pltpu.SemaphoreType.DMA,
        pltpu.SemaphoreType.DMA,
    ],
    compiler_params=pltpu.CompilerParams(use_tc_tiling_on_sc=False),
)(_gather_body)


MM_TOK = 1024


def _mm_body(x_ref, w_ref, b_ref, o_ref):
    x0 = x_ref[0:MM_TOK, :]
    x1 = x_ref[MM_TOK : 2 * MM_TOK, :]
    x = jnp.concatenate([x0, x1], axis=1)
    o_ref[...] = (
        jnp.dot(x, w_ref[...], preferred_element_type=jnp.float32) + b_ref[0, :]
    )


def _tc_matmul(x, w, b):
    return pl.pallas_call(
        _mm_body,
        grid=(N_TOK // MM_TOK,),
        in_specs=[
            pl.BlockSpec((2 * MM_TOK, 128), lambda i: (i, 0)),
            pl.BlockSpec((256, D_EMBED), lambda i: (0, 0)),
            pl.BlockSpec((1, D_EMBED), lambda i: (0, 0)),
        ],
        out_specs=pl.BlockSpec((MM_TOK, D_EMBED), lambda i: (i, 0)),
        out_shape=jax.ShapeDtypeStruct((N_TOK, D_EMBED), jnp.float32),
    )(x, w, b)


@jax.jit
def kernel(input_ids, table, W_trans, b_trans):
    idx_flat = (
        input_ids.reshape(L, 4, 256, 2, 4).transpose(0, 3, 1, 4, 2).reshape(N_ROWS)
    )
    gathered = _sc_gather(idx_flat, table)
    out = _tc_matmul(gathered, W_trans.T, b_trans.reshape(1, D_EMBED))
    return out.reshape(L, B, D_EMBED)


# speedup vs baseline: 16.6978x; 1.1455x over previous
import functools

import jax
import jax.numpy as jnp
from jax import lax
from jax.experimental import pallas as pl
from jax.experimental.pallas import tpu as pltpu
from jax.experimental.pallas import tpu_sc as plsc

L, B, C = 200, 1024, 8
VOCAB, D_SUB, D_EMBED = 100000, 32, 128
N_TOK = L * B
N_ROWS = N_TOK * C
N_OUT = N_ROWS // 4

NC, NS = 2, 16
NW = NC * NS
ROWS_PER_W = N_ROWS // NW
CHUNK = 1024
OUT_CHUNK = CHUNK // 4
N_CHUNKS = ROWS_PER_W // CHUNK


def _writeback(rows_v, out_hbm, out_base):
    for j in range(4):
        pltpu.sync_copy(
            rows_v.at[pl.ds(j * OUT_CHUNK, OUT_CHUNK), :],
            out_hbm.at[pl.ds(out_base, OUT_CHUNK), pl.ds(j * D_SUB, D_SUB)],
        )


def _gather_body(idx_hbm, table_hbm, out_hbm, idx_a, idx_b, rows_a, rows_b,
                 sem_a, sem_b):
    wid = lax.axis_index("s") * NC + lax.axis_index("c")
    base_w = wid * ROWS_PER_W
    n_half = N_CHUNKS // 2

    def load_and_fire(chunk, idx_v, rows_v, sem):
        pltpu.sync_copy(idx_hbm.at[pl.ds(base_w + chunk * CHUNK, CHUNK)], idx_v)
        pltpu.async_copy(table_hbm.at[idx_v], rows_v, sem)

    load_and_fire(0, idx_a, rows_a, sem_a)

    def step(i, carry):
        load_and_fire(2 * i + 1, idx_b, rows_b, sem_b)
        pltpu.make_async_copy(table_hbm.at[idx_a], rows_a, sem_a).wait()
        _writeback(rows_a, out_hbm, (base_w + 2 * i * CHUNK) // 4)

        @pl.when(i < n_half - 1)
        def _():
            load_and_fire(2 * i + 2, idx_a, rows_a, sem_a)

        pltpu.make_async_copy(table_hbm.at[idx_b], rows_b, sem_b).wait()
        _writeback(rows_b, out_hbm, (base_w + (2 * i + 1) * CHUNK) // 4)
        return carry

    lax.fori_loop(0, n_half, step, 0)


_sc_gather = functools.partial(
    pl.kernel,
    out_type=jax.ShapeDtypeStruct((N_OUT, 128), jnp.float32),
    mesh=plsc.VectorSubcoreMesh(core_axis_name="c", subcore_axis_name="s"),
    scratch_types=[
        pltpu.VMEM((CHUNK,), jnp.int32),
        pltpu.VMEM((CHUNK,), jnp.int32),
        pltpu.VMEM((CHUNK, D_SUB), jnp.float32),
        pltpu.VMEM((CHUNK, D_SUB), jnp.float32),
        pltpu.SemaphoreType.DMA,
        pltpu.SemaphoreType.DMA,
    ],
    compiler_params=pltpu.CompilerParams(use_tc_tiling_on_sc=False),
)(_gather_body)


MM_TOK = 1024
MM_GRP = 2


def _mm_body(x_ref, w_ref, b_ref, o_ref):
    for g in range(MM_GRP):
        x0 = x_ref[2 * g * MM_TOK : (2 * g + 1) * MM_TOK, :]
        x1 = x_ref[(2 * g + 1) * MM_TOK : (2 * g + 2) * MM_TOK, :]
        x = jnp.concatenate([x0, x1], axis=1)
        o_ref[g * MM_TOK : (g + 1) * MM_TOK, :] = (
            jnp.dot(x, w_ref[...], preferred_element_type=jnp.float32)
            + b_ref[0, :]
        )


def _tc_matmul(x, w, b):
    return pl.pallas_call(
        _mm_body,
        grid=(N_TOK // (MM_TOK * MM_GRP),),
        in_specs=[
            pl.BlockSpec((2 * MM_TOK * MM_GRP, 128), lambda i: (i, 0)),
            pl.BlockSpec((256, D_EMBED), lambda i: (0, 0)),
            pl.BlockSpec((1, D_EMBED), lambda i: (0, 0)),
        ],
        out_specs=pl.BlockSpec((MM_TOK * MM_GRP, D_EMBED), lambda i: (i, 0)),
        out_shape=jax.ShapeDtypeStruct((N_TOK, D_EMBED), jnp.float32),
    )(x, w, b)


@jax.jit
def kernel(input_ids, table, W_trans, b_trans):
    idx_flat = (
        input_ids.reshape(L, 4, 256, 2, 4).transpose(0, 3, 1, 4, 2).reshape(N_ROWS)
    )
    gathered = _sc_gather(idx_flat, table)
    out = _tc_matmul(gathered, W_trans.T, b_trans.reshape(1, D_EMBED))
    return out.reshape(L, B, D_EMBED)


# speedup vs baseline: 18.1829x; 1.0889x over previous
import functools

import jax
import jax.numpy as jnp
from jax import lax
from jax.experimental import pallas as pl
from jax.experimental.pallas import tpu as pltpu
from jax.experimental.pallas import tpu_sc as plsc

L, B, C = 200, 1024, 8
VOCAB, D_SUB, D_EMBED = 100000, 32, 128
N_TOK = L * B
N_ROWS = N_TOK * C
N_OUT = N_ROWS // 4

NC, NS = 2, 16
NW = NC * NS
ROWS_PER_W = N_ROWS // NW
CHUNK = 1024
OUT_CHUNK = CHUNK // 4
N_CHUNKS = ROWS_PER_W // CHUNK


def _writeback(rows_v, out_hbm, out_base):
    for j in range(4):
        pltpu.sync_copy(
            rows_v.at[pl.ds(j * OUT_CHUNK, OUT_CHUNK), :],
            out_hbm.at[pl.ds(out_base, OUT_CHUNK), pl.ds(j * D_SUB, D_SUB)],
        )


def _gather_body(idx_hbm, table_hbm, out_hbm, idx_a, idx_b, rows_a, rows_b,
                 sem_a, sem_b):
    wid = lax.axis_index("s") * NC + lax.axis_index("c")
    base_w = wid * ROWS_PER_W
    n_half = N_CHUNKS // 2

    def load_and_fire(chunk, idx_v, rows_v, sem):
        pltpu.sync_copy(idx_hbm.at[pl.ds(base_w + chunk * CHUNK, CHUNK)], idx_v)
        pltpu.async_copy(table_hbm.at[idx_v], rows_v, sem)

    load_and_fire(0, idx_a, rows_a, sem_a)

    def step(i, carry):
        load_and_fire(2 * i + 1, idx_b, rows_b, sem_b)
        pltpu.make_async_copy(table_hbm.at[idx_a], rows_a, sem_a).wait()
        _writeback(rows_a, out_hbm, (base_w + 2 * i * CHUNK) // 4)

        @pl.when(i < n_half - 1)
        def _():
            load_and_fire(2 * i + 2, idx_a, rows_a, sem_a)

        pltpu.make_async_copy(table_hbm.at[idx_b], rows_b, sem_b).wait()
        _writeback(rows_b, out_hbm, (base_w + (2 * i + 1) * CHUNK) // 4)
        return carry

    lax.fori_loop(0, n_half, step, 0)


_sc_gather = functools.partial(
    pl.kernel,
    out_type=jax.ShapeDtypeStruct((N_OUT, 128), jnp.float32),
    mesh=plsc.VectorSubcoreMesh(core_axis_name="c", subcore_axis_name="s"),
    scratch_types=[
        pltpu.VMEM((CHUNK,), jnp.int32),
        pltpu.VMEM((CHUNK,), jnp.int32),
        pltpu.VMEM((CHUNK, D_SUB), jnp.float32),
        pltpu.VMEM((CHUNK, D_SUB), jnp.float32),
        pltpu.SemaphoreType.DMA,
        pltpu.SemaphoreType.DMA,
    ],
    compiler_params=pltpu.CompilerParams(use_tc_tiling_on_sc=False),
)(_gather_body)


MM_TOK = 1024
MM_GRP = 4


def _mm_body(x_ref, w_ref, b_ref, o_ref):
    for g in range(MM_GRP):
        x0 = x_ref[2 * g * MM_TOK : (2 * g + 1) * MM_TOK, :]
        x1 = x_ref[(2 * g + 1) * MM_TOK : (2 * g + 2) * MM_TOK, :]
        x = jnp.concatenate([x0, x1], axis=1)
        o_ref[g * MM_TOK : (g + 1) * MM_TOK, :] = (
            jnp.dot(x, w_ref[...], preferred_element_type=jnp.float32)
            + b_ref[0, :]
        )


def _tc_matmul(x, w, b):
    return pl.pallas_call(
        _mm_body,
        grid=(N_TOK // (MM_TOK * MM_GRP),),
        in_specs=[
            pl.BlockSpec((2 * MM_TOK * MM_GRP, 128), lambda i: (i, 0)),
            pl.BlockSpec((256, D_EMBED), lambda i: (0, 0)),
            pl.BlockSpec((1, D_EMBED), lambda i: (0, 0)),
        ],
        out_specs=pl.BlockSpec((MM_TOK * MM_GRP, D_EMBED), lambda i: (i, 0)),
        out_shape=jax.ShapeDtypeStruct((N_TOK, D_EMBED), jnp.float32),
    )(x, w, b)


@jax.jit
def kernel(input_ids, table, W_trans, b_trans):
    idx_flat = (
        input_ids.reshape(L, 4, 256, 2, 4).transpose(0, 3, 1, 4, 2).reshape(N_ROWS)
    )
    gathered = _sc_gather(idx_flat, table)
    out = _tc_matmul(gathered, W_trans.T, b_trans.reshape(1, D_EMBED))
    return out.reshape(L, B, D_EMBED)


# speedup vs baseline: 18.4061x; 1.0123x over previous
import functools

import jax
import jax.numpy as jnp
from jax import lax
from jax.experimental import pallas as pl
from jax.experimental.pallas import tpu as pltpu
from jax.experimental.pallas import tpu_sc as plsc

L, B, C = 200, 1024, 8
VOCAB, D_SUB, D_EMBED = 100000, 32, 128
N_TOK = L * B
N_ROWS = N_TOK * C
N_OUT = N_ROWS // 4

NC, NS = 2, 16
NW = NC * NS
ROWS_PER_W = N_ROWS // NW
CHUNK = 1024
OUT_CHUNK = CHUNK // 4
N_CHUNKS = ROWS_PER_W // CHUNK


def _writeback(rows_v, out_hbm, out_base):
    for j in range(4):
        pltpu.sync_copy(
            rows_v.at[pl.ds(j * OUT_CHUNK, OUT_CHUNK), :],
            out_hbm.at[pl.ds(out_base, OUT_CHUNK), pl.ds(j * D_SUB, D_SUB)],
        )


def _gather_body(idx_hbm, table_hbm, out_hbm, idx_a, idx_b, rows_a, rows_b,
                 sem_a, sem_b):
    wid = lax.axis_index("s") * NC + lax.axis_index("c")
    base_w = wid * ROWS_PER_W
    n_half = N_CHUNKS // 2

    def load_and_fire(chunk, idx_v, rows_v, sem):
        pltpu.sync_copy(idx_hbm.at[pl.ds(base_w + chunk * CHUNK, CHUNK)], idx_v)
        pltpu.async_copy(table_hbm.at[idx_v], rows_v, sem)

    load_and_fire(0, idx_a, rows_a, sem_a)

    def step(i, carry):
        load_and_fire(2 * i + 1, idx_b, rows_b, sem_b)
        pltpu.make_async_copy(table_hbm.at[idx_a], rows_a, sem_a).wait()
        _writeback(rows_a, out_hbm, (base_w + 2 * i * CHUNK) // 4)

        @pl.when(i < n_half - 1)
        def _():
            load_and_fire(2 * i + 2, idx_a, rows_a, sem_a)

        pltpu.make_async_copy(table_hbm.at[idx_b], rows_b, sem_b).wait()
        _writeback(rows_b, out_hbm, (base_w + (2 * i + 1) * CHUNK) // 4)
        return carry

    lax.fori_loop(0, n_half, step, 0)


_sc_gather = functools.partial(
    pl.kernel,
    out_type=jax.ShapeDtypeStruct((N_OUT, 128), jnp.float32),
    mesh=plsc.VectorSubcoreMesh(core_axis_name="c", subcore_axis_name="s"),
    scratch_types=[
        pltpu.VMEM((CHUNK,), jnp.int32),
        pltpu.VMEM((CHUNK,), jnp.int32),
        pltpu.VMEM((CHUNK, D_SUB), jnp.float32),
        pltpu.VMEM((CHUNK, D_SUB), jnp.float32),
        pltpu.SemaphoreType.DMA,
        pltpu.SemaphoreType.DMA,
    ],
    compiler_params=pltpu.CompilerParams(use_tc_tiling_on_sc=False),
)(_gather_body)


MM_TOK = 1024
MM_GRP = 8


def _mm_body(x_ref, w_ref, b_ref, o_ref):
    for g in range(MM_GRP):
        x0 = x_ref[2 * g * MM_TOK : (2 * g + 1) * MM_TOK, :]
        x1 = x_ref[(2 * g + 1) * MM_TOK : (2 * g + 2) * MM_TOK, :]
        x = jnp.concatenate([x0, x1], axis=1)
        o_ref[g * MM_TOK : (g + 1) * MM_TOK, :] = (
            jnp.dot(x, w_ref[...], preferred_element_type=jnp.float32)
            + b_ref[0, :]
        )


def _tc_matmul(x, w, b):
    return pl.pallas_call(
        _mm_body,
        grid=(N_TOK // (MM_TOK * MM_GRP),),
        in_specs=[
            pl.BlockSpec((2 * MM_TOK * MM_GRP, 128), lambda i: (i, 0)),
            pl.BlockSpec((256, D_EMBED), lambda i: (0, 0)),
            pl.BlockSpec((1, D_EMBED), lambda i: (0, 0)),
        ],
        out_specs=pl.BlockSpec((MM_TOK * MM_GRP, D_EMBED), lambda i: (i, 0)),
        out_shape=jax.ShapeDtypeStruct((N_TOK, D_EMBED), jnp.float32),
    )(x, w, b)


@jax.jit
def kernel(input_ids, table, W_trans, b_trans):
    idx_flat = (
        input_ids.reshape(L, 4, 256, 2, 4).transpose(0, 3, 1, 4, 2).reshape(N_ROWS)
    )
    gathered = _sc_gather(idx_flat, table)
    out = _tc_matmul(gathered, W_trans.T, b_trans.reshape(1, D_EMBED))
    return out.reshape(L, B, D_EMBED)


# speedup vs baseline: 18.5058x; 1.0054x over previous
import functools

import jax
import jax.numpy as jnp
from jax import lax
from jax.experimental import pallas as pl
from jax.experimental.pallas import tpu as pltpu
from jax.experimental.pallas import tpu_sc as plsc

L, B, C = 200, 1024, 8
VOCAB, D_SUB, D_EMBED = 100000, 32, 128
N_TOK = L * B
N_ROWS = N_TOK * C
N_OUT = N_ROWS // 4

NC, NS = 2, 16
NW = NC * NS
ROWS_PER_W = N_ROWS // NW
CHUNK = 1024
OUT_CHUNK = CHUNK // 4
N_CHUNKS = ROWS_PER_W // CHUNK


def _gather_body(idx_hbm, table_hbm, out_hbm, idx_a, idx_b, rows_a, rows_b,
                 gsem_a, gsem_b, wsem_a, wsem_b):
    wid = lax.axis_index("s") * NC + lax.axis_index("c")
    base_w = wid * ROWS_PER_W
    n_half = N_CHUNKS // 2

    def load_and_fire(chunk, idx_v, rows_v, gsem):
        pltpu.sync_copy(idx_hbm.at[pl.ds(base_w + chunk * CHUNK, CHUNK)], idx_v)
        pltpu.async_copy(table_hbm.at[idx_v], rows_v, gsem)

    def write_copies(rows_v, out_base, wsem):
        return [
            pltpu.make_async_copy(
                rows_v.at[pl.ds(j * OUT_CHUNK, OUT_CHUNK), :],
                out_hbm.at[pl.ds(out_base, OUT_CHUNK), pl.ds(j * D_SUB, D_SUB)],
                wsem,
            )
            for j in range(4)
        ]

    def fire_writes(rows_v, chunk, wsem):
        for cp in write_copies(rows_v, (base_w + chunk * CHUNK) // 4, wsem):
            cp.start()

    def drain_writes(rows_v, chunk, wsem):
        for cp in write_copies(rows_v, (base_w + chunk * CHUNK) // 4, wsem):
            cp.wait()

    load_and_fire(0, idx_a, rows_a, gsem_a)

    def step(i, carry):
        pltpu.make_async_copy(table_hbm.at[idx_a], rows_a, gsem_a).wait()
        fire_writes(rows_a, 2 * i, wsem_a)

        @pl.when(i > 0)
        def _():
            drain_writes(rows_b, 2 * i - 1, wsem_b)

        load_and_fire(2 * i + 1, idx_b, rows_b, gsem_b)
        pltpu.make_async_copy(table_hbm.at[idx_b], rows_b, gsem_b).wait()
        fire_writes(rows_b, 2 * i + 1, wsem_b)

        @pl.when(i < n_half - 1)
        def _():
            drain_writes(rows_a, 2 * i, wsem_a)
            load_and_fire(2 * i + 2, idx_a, rows_a, gsem_a)

        return carry

    lax.fori_loop(0, n_half, step, 0)
    drain_writes(rows_a, N_CHUNKS - 2, wsem_a)
    drain_writes(rows_b, N_CHUNKS - 1, wsem_b)


_sc_gather = functools.partial(
    pl.kernel,
    out_type=jax.ShapeDtypeStruct((N_OUT, 128), jnp.float32),
    mesh=plsc.VectorSubcoreMesh(core_axis_name="c", subcore_axis_name="s"),
    scratch_types=[
        pltpu.VMEM((CHUNK,), jnp.int32),
        pltpu.VMEM((CHUNK,), jnp.int32),
        pltpu.VMEM((CHUNK, D_SUB), jnp.float32),
        pltpu.VMEM((CHUNK, D_SUB), jnp.float32),
        pltpu.SemaphoreType.DMA,
        pltpu.SemaphoreType.DMA,
        pltpu.SemaphoreType.DMA,
        pltpu.SemaphoreType.DMA,
    ],
    compiler_params=pltpu.CompilerParams(use_tc_tiling_on_sc=False),
)(_gather_body)


MM_TOK = 1024
MM_GRP = 8


def _mm_body(x_ref, w_ref, b_ref, o_ref):
    for g in range(MM_GRP):
        x0 = x_ref[2 * g * MM_TOK : (2 * g + 1) * MM_TOK, :]
        x1 = x_ref[(2 * g + 1) * MM_TOK : (2 * g + 2) * MM_TOK, :]
        x = jnp.concatenate([x0, x1], axis=1)
        o_ref[g * MM_TOK : (g + 1) * MM_TOK, :] = (
            jnp.dot(x, w_ref[...], preferred_element_type=jnp.float32)
            + b_ref[0, :]
        )


def _tc_matmul(x, w, b):
    return pl.pallas_call(
        _mm_body,
        grid=(N_TOK // (MM_TOK * MM_GRP),),
        in_specs=[
            pl.BlockSpec((2 * MM_TOK * MM_GRP, 128), lambda i: (i, 0)),
            pl.BlockSpec((256, D_EMBED), lambda i: (0, 0)),
            pl.BlockSpec((1, D_EMBED), lambda i: (0, 0)),
        ],
        out_specs=pl.BlockSpec((MM_TOK * MM_GRP, D_EMBED), lambda i: (i, 0)),
        out_shape=jax.ShapeDtypeStruct((N_TOK, D_EMBED), jnp.float32),
    )(x, w, b)


@jax.jit
def kernel(input_ids, table, W_trans, b_trans):
    idx_flat = (
        input_ids.reshape(L, 4, 256, 2, 4).transpose(0, 3, 1, 4, 2).reshape(N_ROWS)
    )
    gathered = _sc_gather(idx_flat, table)
    out = _tc_matmul(gathered, W_trans.T, b_trans.reshape(1, D_EMBED))
    return out.reshape(L, B, D_EMBED)


# speedup vs baseline: 19.1172x; 1.0330x over previous
import functools

import jax
import jax.numpy as jnp
from jax import lax
from jax.experimental import pallas as pl
from jax.experimental.pallas import tpu as pltpu
from jax.experimental.pallas import tpu_sc as plsc

L, B, C = 200, 1024, 8
VOCAB, D_SUB, D_EMBED = 100000, 32, 128
N_TOK = L * B
N_ROWS = N_TOK * C

N_SEG = 5
L_SEG = L // N_SEG
SEG_ROWS = N_ROWS // N_SEG
SEG_OUT = SEG_ROWS // 4

NC, NS = 2, 16
NW = NC * NS
ROWS_PER_W = SEG_ROWS // NW
CHUNK = 1024
OUT_CHUNK = CHUNK // 4
N_CHUNKS = ROWS_PER_W // CHUNK


def _gather_body(idx_hbm, table_hbm, out_hbm, idx_a, idx_b, rows_a, rows_b,
                 gsem_a, gsem_b, wsem_a, wsem_b):
    wid = lax.axis_index("s") * NC + lax.axis_index("c")
    base_w = wid * ROWS_PER_W
    n_half = N_CHUNKS // 2

    def load_and_fire(chunk, idx_v, rows_v, gsem):
        pltpu.sync_copy(idx_hbm.at[pl.ds(base_w + chunk * CHUNK, CHUNK)], idx_v)
        pltpu.async_copy(table_hbm.at[idx_v], rows_v, gsem)

    def write_copies(rows_v, out_base, wsem):
        return [
            pltpu.make_async_copy(
                rows_v.at[pl.ds(j * OUT_CHUNK, OUT_CHUNK), :],
                out_hbm.at[pl.ds(out_base, OUT_CHUNK), pl.ds(j * D_SUB, D_SUB)],
                wsem,
            )
            for j in range(4)
        ]

    def fire_writes(rows_v, chunk, wsem):
        for cp in write_copies(rows_v, (base_w + chunk * CHUNK) // 4, wsem):
            cp.start()

    def drain_writes(rows_v, chunk, wsem):
        for cp in write_copies(rows_v, (base_w + chunk * CHUNK) // 4, wsem):
            cp.wait()

    load_and_fire(0, idx_a, rows_a, gsem_a)

    def step(i, carry):
        pltpu.make_async_copy(table_hbm.at[idx_a], rows_a, gsem_a).wait()
        fire_writes(rows_a, 2 * i, wsem_a)

        @pl.when(i > 0)
        def _():
            drain_writes(rows_b, 2 * i - 1, wsem_b)

        load_and_fire(2 * i + 1, idx_b, rows_b, gsem_b)
        pltpu.make_async_copy(table_hbm.at[idx_b], rows_b, gsem_b).wait()
        fire_writes(rows_b, 2 * i + 1, wsem_b)

        @pl.when(i < n_half - 1)
        def _():
            drain_writes(rows_a, 2 * i, wsem_a)
            load_and_fire(2 * i + 2, idx_a, rows_a, gsem_a)

        return carry

    lax.fori_loop(0, n_half, step, 0)
    drain_writes(rows_a, N_CHUNKS - 2, wsem_a)
    drain_writes(rows_b, N_CHUNKS - 1, wsem_b)


_sc_gather = functools.partial(
    pl.kernel,
    out_type=jax.ShapeDtypeStruct((SEG_OUT, 128), jnp.float32),
    mesh=plsc.VectorSubcoreMesh(core_axis_name="c", subcore_axis_name="s"),
    scratch_types=[
        pltpu.VMEM((CHUNK,), jnp.int32),
        pltpu.VMEM((CHUNK,), jnp.int32),
        pltpu.VMEM((CHUNK, D_SUB), jnp.float32),
        pltpu.VMEM((CHUNK, D_SUB), jnp.float32),
        pltpu.SemaphoreType.DMA,
        pltpu.SemaphoreType.DMA,
        pltpu.SemaphoreType.DMA,
        pltpu.SemaphoreType.DMA,
    ],
    compiler_params=pltpu.CompilerParams(use_tc_tiling_on_sc=False),
)(_gather_body)


MM_TOK = 1024
MM_GRP = 5
SEG_BLOCKS = L_SEG // MM_GRP


def _mm_compute(x_ref, w_ref, b_ref, o_ref):
    for g in range(MM_GRP):
        x0 = x_ref[2 * g * MM_TOK : (2 * g + 1) * MM_TOK, :]
        x1 = x_ref[(2 * g + 1) * MM_TOK : (2 * g + 2) * MM_TOK, :]
        x = jnp.concatenate([x0, x1], axis=1)
        o_ref[g * MM_TOK : (g + 1) * MM_TOK, :] = (
            jnp.dot(x, w_ref[...], preferred_element_type=jnp.float32)
            + b_ref[0, :]
        )


def _mm_body_first(x_ref, w_ref, b_ref, o_ref):
    _mm_compute(x_ref, w_ref, b_ref, o_ref)


def _mm_body_chained(x_ref, w_ref, b_ref, acc_ref, o_ref):
    del acc_ref
    _mm_compute(x_ref, w_ref, b_ref, o_ref)


def _tc_matmul_seg(seg, x, w, b, acc=None):
    blk = MM_TOK * MM_GRP

    def out_map(i, s=seg):
        return (s * SEG_BLOCKS + i, 0)

    in_specs = [
        pl.BlockSpec((2 * blk, 128), lambda i: (i, 0)),
        pl.BlockSpec((256, D_EMBED), lambda i: (0, 0)),
        pl.BlockSpec((1, D_EMBED), lambda i: (0, 0)),
    ]
    args = (x, w, b)
    if acc is None:
        body = _mm_body_first
        aliases = {}
    else:
        body = _mm_body_chained
        in_specs = in_specs + [pl.BlockSpec(memory_space=pl.ANY)]
        args = args + (acc,)
        aliases = {3: 0}
    return pl.pallas_call(
        body,
        grid=(SEG_BLOCKS,),
        in_specs=in_specs,
        out_specs=pl.BlockSpec((blk, D_EMBED), out_map),
        out_shape=jax.ShapeDtypeStruct((N_TOK, D_EMBED), jnp.float32),
        input_output_aliases=aliases,
    )(*args)


@jax.jit
def kernel(input_ids, table, W_trans, b_trans):
    wt = W_trans.T
    bias = b_trans.reshape(1, D_EMBED)
    stagings = []
    for s in range(N_SEG):
        ids_s = input_ids[s * L_SEG : (s + 1) * L_SEG]
        idx_flat = (
            ids_s.reshape(L_SEG, 4, 256, 2, 4)
            .transpose(0, 3, 1, 4, 2)
            .reshape(SEG_ROWS)
        )
        stagings.append(_sc_gather(idx_flat, table))
    out = None
    for s in range(N_SEG):
        out = _tc_matmul_seg(s, stagings[s], wt, bias, acc=out)
    return out.reshape(L, B, D_EMBED)
